# Initial kernel scaffold; baseline (speedup 1.0000x reference)
import jax
import jax.numpy as jnp
from jax.experimental import pallas as pl


def kernel(x, W1, b1, W2, b2, PW1, Pb1, PW2, Pb2, edge_index):
    raise NotImplementedError("write your pallas kernel here")


# speedup vs baseline: 12.4021x; 12.4021x over previous
import functools

import jax
import jax.numpy as jnp
from jax import lax
from jax.experimental import pallas as pl
from jax.experimental.pallas import tpu as pltpu
from jax.experimental.pallas import tpu_sc as plsc

N_CORES = 2
N_SUB = 16
NW = N_CORES * N_SUB
CHUNK = 128
DEG_W = 128


def _sc_degree(dst_t, ones_blk, zeros_blk, n_pad):
    c_chunks = dst_t.shape[1]
    rpt = n_pad // N_SUB
    zsteps = rpt // CHUNK
    mesh = plsc.VectorSubcoreMesh(core_axis_name="c", subcore_axis_name="s",
                                  num_cores=N_CORES, num_subcores=N_SUB)

    @functools.partial(
        pl.kernel,
        out_type=jax.ShapeDtypeStruct((N_CORES, n_pad, DEG_W), jnp.float32),
        mesh=mesh,
        scratch_types=[
            pltpu.VMEM((c_chunks, CHUNK), jnp.int32),
            pltpu.VMEM((CHUNK, DEG_W), jnp.float32),
            pltpu.VMEM((CHUNK, DEG_W), jnp.float32),
            pltpu.VMEM_SHARED((n_pad, DEG_W), jnp.float32),
        ],
    )
    def k(dst_hbm, ones_hbm, zeros_hbm, out_hbm, dstv, onesv, buf, deg_sh):
        cid = lax.axis_index("c")
        sid = lax.axis_index("s")
        wid = sid * N_CORES + cid
        base = sid * rpt
        pltpu.sync_copy(zeros_hbm, buf)

        @pl.loop(0, zsteps)
        def _zero(t):
            pltpu.sync_copy(buf, deg_sh.at[pl.ds(base + t * CHUNK, CHUNK)])

        pltpu.sync_copy(ones_hbm, onesv)
        pltpu.sync_copy(dst_hbm.at[wid], dstv)
        plsc.subcore_barrier()

        @pl.loop(0, c_chunks)
        def _edges(j):
            pltpu.sync_copy(onesv, deg_sh.at[dstv.at[j]], add=True)

        plsc.subcore_barrier()

        @pl.loop(0, zsteps)
        def _out(t):
            pltpu.sync_copy(deg_sh.at[pl.ds(base + t * CHUNK, CHUNK)], buf)
            pltpu.sync_copy(
                buf, out_hbm.at[cid, pl.ds(base + t * CHUNK, CHUNK)])

    return k(dst_t, ones_blk, zeros_blk)


def _sc_agg(y, src_t, dst_t, zeros_blk, n_pad):
    c_chunks = src_t.shape[1]
    d = y.shape[1]
    rpt = n_pad // N_SUB
    zsteps = rpt // CHUNK
    mesh = plsc.VectorSubcoreMesh(core_axis_name="c", subcore_axis_name="s",
                                  num_cores=N_CORES, num_subcores=N_SUB)

    @functools.partial(
        pl.kernel,
        out_type=jax.ShapeDtypeStruct((N_CORES, n_pad, d), jnp.float32),
        mesh=mesh,
        scratch_types=[
            pltpu.VMEM((c_chunks, CHUNK), jnp.int32),
            pltpu.VMEM((c_chunks, CHUNK), jnp.int32),
            pltpu.VMEM((CHUNK, d), jnp.float32),
            pltpu.VMEM_SHARED((n_pad, d), jnp.float32),
            pltpu.SemaphoreType.DMA,
        ],
    )
    def k(y_hbm, src_hbm, dst_hbm, zeros_hbm, out_hbm,
          srcv, dstv, rows, agg_sh, sem):
        cid = lax.axis_index("c")
        sid = lax.axis_index("s")
        wid = sid * N_CORES + cid
        base = sid * rpt
        pltpu.sync_copy(zeros_hbm, rows)

        @pl.loop(0, zsteps)
        def _zero(t):
            pltpu.sync_copy(rows, agg_sh.at[pl.ds(base + t * CHUNK, CHUNK)])

        pltpu.sync_copy(src_hbm.at[wid], srcv)
        pltpu.sync_copy(dst_hbm.at[wid], dstv)
        plsc.subcore_barrier()

        @pl.loop(0, c_chunks)
        def _edges(j):
            pltpu.async_copy(y_hbm.at[srcv.at[j]], rows, sem).wait()
            pltpu.sync_copy(rows, agg_sh.at[dstv.at[j]], add=True)

        plsc.subcore_barrier()

        @pl.loop(0, zsteps)
        def _out(t):
            pltpu.sync_copy(agg_sh.at[pl.ds(base + t * CHUNK, CHUNK)], rows)
            pltpu.sync_copy(
                rows, out_hbm.at[cid, pl.ds(base + t * CHUNK, CHUNK)])

    return k(y, src_t, dst_t, zeros_blk)


def _dinv_of(deg_ref):
    cnt = (deg_ref[0] + deg_ref[1])[:, 0:1]
    return lax.rsqrt(cnt + 1.0)


def _tc_prescale(x_pad, w, degp, blk):
    n_pad = x_pad.shape[0]

    def body(x_ref, w_ref, deg_ref, y_ref):
        dinv = _dinv_of(deg_ref)
        xw = jnp.dot(x_ref[...], w_ref[...],
                     preferred_element_type=jnp.float32)
        y_ref[...] = xw * dinv

    return pl.pallas_call(
        body,
        grid=(n_pad // blk,),
        in_specs=[
            pl.BlockSpec((blk, 128), lambda i: (i, 0)),
            pl.BlockSpec((128, 128), lambda i: (0, 0)),
            pl.BlockSpec((N_CORES, blk, DEG_W), lambda i: (0, i, 0)),
        ],
        out_specs=pl.BlockSpec((blk, 128), lambda i: (i, 0)),
        out_shape=jax.ShapeDtypeStruct((n_pad, 128), jnp.float32),
    )(x_pad, w, degp)


def _tc_mid(aggp, y1, degp, b1, w2, blk):
    n_pad = y1.shape[0]

    def body(agg_ref, y1_ref, deg_ref, b1_ref, w2_ref, y2_ref):
        dinv = _dinv_of(deg_ref)
        pre = dinv * (agg_ref[0] + agg_ref[1] + y1_ref[...]) + b1_ref[...]
        h = jnp.maximum(pre, 0.0)
        y2_ref[...] = jnp.dot(h, w2_ref[...],
                              preferred_element_type=jnp.float32) * dinv

    return pl.pallas_call(
        body,
        grid=(n_pad // blk,),
        in_specs=[
            pl.BlockSpec((N_CORES, blk, 128), lambda i: (0, i, 0)),
            pl.BlockSpec((blk, 128), lambda i: (i, 0)),
            pl.BlockSpec((N_CORES, blk, DEG_W), lambda i: (0, i, 0)),
            pl.BlockSpec((1, 128), lambda i: (0, 0)),
            pl.BlockSpec((128, 128), lambda i: (0, 0)),
        ],
        out_specs=pl.BlockSpec((blk, 128), lambda i: (i, 0)),
        out_shape=jax.ShapeDtypeStruct((n_pad, 128), jnp.float32),
    )(aggp, y1, degp, b1, w2)


def _tc_head(aggp, y2, degp, b2, pw1, pb1, pw2, pb2, blk):
    n_pad = y2.shape[0]

    def body(agg_ref, y2_ref, deg_ref, b2_ref, pw1_ref, pb1_ref,
             pw2_ref, pb2_ref, z_ref, zp_ref):
        dinv = _dinv_of(deg_ref)
        z = dinv * (agg_ref[0] + agg_ref[1] + y2_ref[...]) + b2_ref[...]
        z_ref[...] = z
        t = jnp.dot(z, pw1_ref[...],
                    preferred_element_type=jnp.float32) + pb1_ref[...]
        zp = jnp.where(t > 0.0, t, jnp.exp(t) - 1.0)
        zp_ref[...] = jnp.dot(zp, pw2_ref[...],
                              preferred_element_type=jnp.float32) + pb2_ref[...]

    return pl.pallas_call(
        body,
        grid=(n_pad // blk,),
        in_specs=[
            pl.BlockSpec((N_CORES, blk, 128), lambda i: (0, i, 0)),
            pl.BlockSpec((blk, 128), lambda i: (i, 0)),
            pl.BlockSpec((N_CORES, blk, DEG_W), lambda i: (0, i, 0)),
            pl.BlockSpec((1, 128), lambda i: (0, 0)),
            pl.BlockSpec((128, 128), lambda i: (0, 0)),
            pl.BlockSpec((1, 128), lambda i: (0, 0)),
            pl.BlockSpec((128, 128), lambda i: (0, 0)),
            pl.BlockSpec((1, 128), lambda i: (0, 0)),
        ],
        out_specs=[
            pl.BlockSpec((blk, 128), lambda i: (i, 0)),
            pl.BlockSpec((blk, 128), lambda i: (i, 0)),
        ],
        out_shape=[
            jax.ShapeDtypeStruct((n_pad, 128), jnp.float32),
            jax.ShapeDtypeStruct((n_pad, 128), jnp.float32),
        ],
    )(aggp, y2, degp, b2, pw1, pb1, pw2, pb2)


def kernel(x, W1, b1, W2, b2, PW1, Pb1, PW2, Pb2, edge_index):
    n, d = x.shape
    e = edge_index.shape[1]

    node_blk = N_SUB * CHUNK
    n_pad = -(-n // node_blk) * node_blk
    e_blk = NW * CHUNK
    c_chunks = -(-e // e_blk)
    e_pad = c_chunks * e_blk

    src = jnp.concatenate(
        [edge_index[0], jnp.zeros((e_pad - e,), jnp.int32)])
    dst = jnp.concatenate(
        [edge_index[1], jnp.full((e_pad - e,), n, jnp.int32)])
    src_t = src.reshape(NW, c_chunks, CHUNK)
    dst_t = dst.reshape(NW, c_chunks, CHUNK)

    x_pad = jnp.pad(x, ((0, n_pad - n), (0, 0)))
    ones_blk = jnp.ones((CHUNK, DEG_W), jnp.float32)
    zeros_deg = jnp.zeros((CHUNK, DEG_W), jnp.float32)
    zeros_row = jnp.zeros((CHUNK, d), jnp.float32)

    degp = _sc_degree(dst_t, ones_blk, zeros_deg, n_pad)

    blk = 1024
    y1 = _tc_prescale(x_pad, W1, degp, blk)
    agg1 = _sc_agg(y1, src_t, dst_t, zeros_row, n_pad)
    y2 = _tc_mid(agg1, y1, degp, b1.reshape(1, 128), W2, blk)
    agg2 = _sc_agg(y2, src_t, dst_t, zeros_row, n_pad)
    z, z_proj = _tc_head(agg2, y2, degp, b2.reshape(1, 128),
                         PW1, Pb1.reshape(1, 128), PW2, Pb2.reshape(1, 128),
                         blk)
    return z[:n], z_proj[:n]
